# baseline (speedup 1.0000x reference)
import functools

import jax
import jax.numpy as jnp
from jax import lax
from jax.experimental import pallas as pl
from jax.experimental.pallas import tpu as pltpu
from jax.experimental.pallas import tpu_sc as plsc

_N = 10000
_E = 320000
_B = 2
_F = 128
_NT = 16
_NC = 2
_CHUNK = _E // _NT
_HALF = _CHUNK // _NC
_PIECE = 2000
_NPP = _HALF // _PIECE
_NPAD = 10240
_SLICE = _NPAD // _NT

_SC_PARAMS = pltpu.CompilerParams(use_tc_tiling_on_sc=False,
                                  needs_layout_passes=False)
_SC_MESH = plsc.VectorSubcoreMesh(core_axis_name="c", subcore_axis_name="s")


def _rsqrt16(d):
    i = plsc.bitcast(d, jnp.int32)
    i = jnp.int32(0x5F3759DF) - (i >> 1)
    y = plsc.bitcast(i, jnp.float32)
    for _ in range(3):
        y = y * (1.5 - 0.5 * d * y * y)
    return jnp.where(d > 0.0, y, 0.0)


def _piece_starter(ei_hbm, w_hbm, bufs):
    def start_piece(q, off):
        sv, dv, wv, sem = bufs[q]
        cps = (pltpu.make_async_copy(ei_hbm.at[0, pl.ds(off, _PIECE)], sv, sem),
               pltpu.make_async_copy(ei_hbm.at[1, pl.ds(off, _PIECE)], dv, sem),
               pltpu.make_async_copy(w_hbm.at[pl.ds(off, _PIECE)], wv, sem))
        for cp in cps:
            cp.start()
        return cps
    return start_piece


def _deg_body(ei_hbm, w_hbm, degp_hbm,
              sA, dA, wA, sB, dB, wB, deg_v, red_v, slc_v, semA, semB,
              deg_sh):
    c = lax.axis_index("c")
    s = lax.axis_index("s")
    bufs = ((sA, dA, wA, semA), (sB, dB, wB, semB))
    start_piece = _piece_starter(ei_hbm, w_hbm, bufs)
    base = s * _CHUNK + c * _HALF

    pend = start_piece(0, base)

    def _zero(i, carry):
        for u in range(5):
            deg_v[pl.ds(i * 80 + u * 16, 16)] = jnp.zeros((16,), jnp.float32)
        return carry
    lax.fori_loop(0, _N // 80, _zero, None)

    def _make_deg(q):
        sv, dv, wv, _ = bufs[q]

        def _deg(i, carry):
            for u in range(5):
                off = i * 80 + u * 16
                a = sv[pl.ds(off, 16)]
                b = dv[pl.ds(off, 16)]
                w16 = wv[pl.ds(off, 16)]
                plsc.addupdate_scatter(deg_v, [a], jnp.where(a != b, w16, 0.0))
            return carry
        return _deg

    degs = (_make_deg(0), _make_deg(1))
    for p in range(_NPP):
        q = p & 1
        cur = pend
        if p + 1 < _NPP:
            pend = start_piece(q ^ 1, base + (p + 1) * _PIECE)
        for cp in cur:
            cp.wait()
        lax.fori_loop(0, _PIECE // 80, degs[q], None)

    pltpu.sync_copy(deg_v, deg_sh.at[s, pl.ds(0, _N)])
    plsc.subcore_barrier()

    pltpu.sync_copy(deg_sh.at[:, pl.ds(s * _SLICE, _SLICE)], red_v)

    def _dred(j, carry):
        acc = red_v[0, pl.ds(j * 16, 16)]
        for k in range(1, _NT):
            acc = acc + red_v[k, pl.ds(j * 16, 16)]
        slc_v[pl.ds(j * 16, 16)] = acc
        return carry
    lax.fori_loop(0, _SLICE // 16, _dred, None)
    pltpu.sync_copy(slc_v, degp_hbm.at[c, pl.ds(s * _SLICE, _SLICE)])


_deg_sc = functools.partial(
    pl.kernel,
    out_type=jax.ShapeDtypeStruct((_NC, _NPAD), jnp.float32),
    mesh=_SC_MESH,
    compiler_params=_SC_PARAMS,
    scratch_types=[
        pltpu.VMEM((_PIECE,), jnp.int32),
        pltpu.VMEM((_PIECE,), jnp.int32),
        pltpu.VMEM((_PIECE,), jnp.float32),
        pltpu.VMEM((_PIECE,), jnp.int32),
        pltpu.VMEM((_PIECE,), jnp.int32),
        pltpu.VMEM((_PIECE,), jnp.float32),
        pltpu.VMEM((_N,), jnp.float32),
        pltpu.VMEM((_NT, _SLICE), jnp.float32),
        pltpu.VMEM((_SLICE,), jnp.float32),
        pltpu.SemaphoreType.DMA,
        pltpu.SemaphoreType.DMA,
        pltpu.VMEM_SHARED((_NT, _NPAD), jnp.float32),
    ],
)(_deg_body)


def _msg_body(ei_hbm, w_hbm, z1_hbm, degp_hbm, t_hbm,
              sA, dA, wA, sB, dB, wB, z1_v, dinv_v, t_v, red_v, slc_v, dp_v,
              z1_sem, semA, semB, dinv_sh, t_sh):
    c = lax.axis_index("c")
    s = lax.axis_index("s")
    bufs = ((sA, dA, wA, semA), (sB, dB, wB, semB))
    start_piece = _piece_starter(ei_hbm, w_hbm, bufs)
    base = s * _CHUNK + c * _HALF

    pend = start_piece(0, base)
    z1_cp = pltpu.make_async_copy(z1_hbm, z1_v, z1_sem)
    z1_cp.start()

    pltpu.sync_copy(degp_hbm.at[:, pl.ds(s * _SLICE, _SLICE)], dp_v)

    def _dinv(j, carry):
        acc = dp_v[0, pl.ds(j * 16, 16)] + dp_v[1, pl.ds(j * 16, 16)]
        slc_v[pl.ds(j * 16, 16)] = _rsqrt16(acc)
        return carry
    lax.fori_loop(0, _SLICE // 16, _dinv, None)
    pltpu.sync_copy(slc_v, dinv_sh.at[pl.ds(s * _SLICE, _SLICE)])

    def _zero(i, carry):
        for u in range(5):
            t_v[pl.ds(i * 80 + u * 16, 16)] = jnp.zeros((16,), jnp.float32)
        return carry
    lax.fori_loop(0, (_B * _N) // 80, _zero, None)

    plsc.subcore_barrier()
    pltpu.sync_copy(dinv_sh.at[pl.ds(0, _N)], dinv_v)
    z1_cp.wait()

    def _make_msg(q):
        sv, dv, wv, _ = bufs[q]

        def _msg(i, carry):
            for u in range(5):
                off = i * 80 + u * 16
                a = sv[pl.ds(off, 16)]
                b = dv[pl.ds(off, 16)]
                w16 = wv[pl.ds(off, 16)]
                dsrc = plsc.load_gather(dinv_v, [a])
                ddst = plsc.load_gather(dinv_v, [b])
                ce = jnp.where(a != b, -(dsrc * w16 * ddst), 0.0)
                g0 = plsc.load_gather(z1_v, [a])
                g1 = plsc.load_gather(z1_v, [a + _N])
                plsc.addupdate_scatter(t_v, [b], ce * g0)
                plsc.addupdate_scatter(t_v, [b + _N], ce * g1)
            return carry
        return _msg

    msgs = (_make_msg(0), _make_msg(1))
    for p in range(_NPP):
        q = p & 1
        cur = pend
        if p + 1 < _NPP:
            pend = start_piece(q ^ 1, base + (p + 1) * _PIECE)
        for cp in cur:
            cp.wait()
        lax.fori_loop(0, _PIECE // 80, msgs[q], None)

    pltpu.sync_copy(t_v.at[pl.ds(0, _N)], t_sh.at[s, 0, pl.ds(0, _N)])
    pltpu.sync_copy(t_v.at[pl.ds(_N, _N)], t_sh.at[s, 1, pl.ds(0, _N)])
    plsc.subcore_barrier()

    for b in range(_B):
        pltpu.sync_copy(t_sh.at[:, b, pl.ds(s * _SLICE, _SLICE)], red_v)

        def _tred(j, carry):
            acc = red_v[0, pl.ds(j * 16, 16)]
            for k in range(1, _NT):
                acc = acc + red_v[k, pl.ds(j * 16, 16)]
            slc_v[pl.ds(j * 16, 16)] = acc
            return carry
        lax.fori_loop(0, _SLICE // 16, _tred, None)
        pltpu.sync_copy(slc_v, t_hbm.at[b, c, pl.ds(s * _SLICE, _SLICE)])


_msg_sc = functools.partial(
    pl.kernel,
    out_type=jax.ShapeDtypeStruct((_B, _NC, _NPAD), jnp.float32),
    mesh=_SC_MESH,
    compiler_params=_SC_PARAMS,
    scratch_types=[
        pltpu.VMEM((_PIECE,), jnp.int32),
        pltpu.VMEM((_PIECE,), jnp.int32),
        pltpu.VMEM((_PIECE,), jnp.float32),
        pltpu.VMEM((_PIECE,), jnp.int32),
        pltpu.VMEM((_PIECE,), jnp.int32),
        pltpu.VMEM((_PIECE,), jnp.float32),
        pltpu.VMEM((_B * _N,), jnp.float32),
        pltpu.VMEM((_N,), jnp.float32),
        pltpu.VMEM((_B * _N,), jnp.float32),
        pltpu.VMEM((_NT, _SLICE), jnp.float32),
        pltpu.VMEM((_SLICE,), jnp.float32),
        pltpu.VMEM((_NC, _SLICE), jnp.float32),
        pltpu.SemaphoreType.DMA,
        pltpu.SemaphoreType.DMA,
        pltpu.SemaphoreType.DMA,
        pltpu.VMEM_SHARED((_NPAD,), jnp.float32),
        pltpu.VMEM_SHARED((_NT, _B, _NPAD), jnp.float32),
    ],
)(_msg_body)


def _mv_body(x_ref, w0_ref, w1_ref, z_ref):
    xb = x_ref[...]
    z0 = jnp.dot(xb, w0_ref[...], preferred_element_type=jnp.float32)
    z1 = jnp.dot(xb, w1_ref[...], preferred_element_type=jnp.float32)
    z_ref[...] = jnp.concatenate([z0, z1], axis=1)


def _matvec(x2, w0, w1):
    return pl.pallas_call(
        _mv_body,
        grid=(5,),
        in_specs=[
            pl.BlockSpec((4000, _F), lambda i: (i, 0)),
            pl.BlockSpec((_F, 1), lambda i: (0, 0)),
            pl.BlockSpec((_F, 1), lambda i: (0, 0)),
        ],
        out_specs=pl.BlockSpec((4000, 2), lambda i: (i, 0)),
        out_shape=jax.ShapeDtypeStruct((_B * _N, 2), jnp.float32),
    )(x2, w0, w1)


def _y_body(z0_ref, t_ref, b_ref, y_ref):
    sv = z0_ref[0, 0] + t_ref[0, 0, :_N] + t_ref[0, 1, :_N] + b_ref[0]
    y_ref[0, 0] = jax.nn.sigmoid(sv)


def _sigmoid_y(z0, t, bias):
    return pl.pallas_call(
        _y_body,
        grid=(_B,),
        in_specs=[
            pl.BlockSpec((1, 1, _N), lambda b: (b, 0, 0)),
            pl.BlockSpec((1, _NC, _NPAD), lambda b: (b, 0, 0)),
            pl.BlockSpec((1,), lambda b: (0,)),
        ],
        out_specs=pl.BlockSpec((1, 1, _N), lambda b: (b, 0, 0)),
        out_shape=jax.ShapeDtypeStruct((_B, 1, _N), jnp.float32),
    )(z0, t, bias)


def kernel(x, edge_index, edge_w, W0, W1, bias):
    ei = edge_index.astype(jnp.int32)
    x2 = x.reshape(_B * _N, _F)
    degp = _deg_sc(ei, edge_w)
    z = _matvec(x2, W0, W1)
    t = _msg_sc(ei, edge_w, z[:, 1], degp)
    y = _sigmoid_y(z[:, 0].reshape(_B, 1, _N), t, bias)
    one = 1.0 + 0.0 * bias[0]
    return jnp.concatenate([x * one, y.reshape(_B, _N, 1)], axis=-1)


# speedup vs baseline: 2.1200x; 2.1200x over previous
import functools

import jax
import jax.numpy as jnp
from jax import lax
from jax.experimental import pallas as pl
from jax.experimental.pallas import tpu as pltpu
from jax.experimental.pallas import tpu_sc as plsc

_N = 10000
_E = 320000
_B = 2
_F = 128
_NT = 16
_NC = 2
_CHUNK = _E // _NT
_HALF = _CHUNK // _NC
_PIECE = 2000
_NPP = _HALF // _PIECE
_NPAD = 10240
_SLICE = _NPAD // _NT

_SC_PARAMS = pltpu.CompilerParams(use_tc_tiling_on_sc=False,
                                  needs_layout_passes=False)
_SC_MESH = plsc.VectorSubcoreMesh(core_axis_name="c", subcore_axis_name="s")


def _rsqrt16(d):
    i = plsc.bitcast(d, jnp.int32)
    i = jnp.int32(0x5F3759DF) - (i >> 1)
    y = plsc.bitcast(i, jnp.float32)
    for _ in range(3):
        y = y * (1.5 - 0.5 * d * y * y)
    return jnp.where(d > 0.0, y, 0.0)


def _piece_starter(ei_hbm, w_hbm, bufs):
    def start_piece(q, off):
        sv, dv, wv, sem = bufs[q]
        cps = (pltpu.make_async_copy(ei_hbm.at[0, pl.ds(off, _PIECE)], sv, sem),
               pltpu.make_async_copy(ei_hbm.at[1, pl.ds(off, _PIECE)], dv, sem),
               pltpu.make_async_copy(w_hbm.at[pl.ds(off, _PIECE)], wv, sem))
        for cp in cps:
            cp.start()
        return cps
    return start_piece


def _deg_body(ei_hbm, w_hbm, degp_hbm,
              sA, dA, wA, sB, dB, wB, deg_v, red_v, slc_v, semA, semB,
              deg_sh):
    c = lax.axis_index("c")
    s = lax.axis_index("s")
    bufs = ((sA, dA, wA, semA), (sB, dB, wB, semB))
    start_piece = _piece_starter(ei_hbm, w_hbm, bufs)
    base = s * _CHUNK + c * _HALF

    pend = start_piece(0, base)

    def _zero(i, carry):
        for u in range(5):
            deg_v[pl.ds(i * 80 + u * 16, 16)] = jnp.zeros((16,), jnp.float32)
        return carry
    lax.fori_loop(0, _N // 80, _zero, None)

    def _make_deg(q):
        sv, dv, wv, _ = bufs[q]

        def _deg(i, carry):
            for u in range(5):
                off = i * 80 + u * 16
                a = sv[pl.ds(off, 16)]
                b = dv[pl.ds(off, 16)]
                w16 = wv[pl.ds(off, 16)]
                plsc.addupdate_scatter(deg_v, [a], jnp.where(a != b, w16, 0.0))
            return carry
        return _deg

    degs = (_make_deg(0), _make_deg(1))
    for p in range(_NPP):
        q = p & 1
        cur = pend
        if p + 1 < _NPP:
            pend = start_piece(q ^ 1, base + (p + 1) * _PIECE)
        for cp in cur:
            cp.wait()
        lax.fori_loop(0, _PIECE // 80, degs[q], None)

    pltpu.sync_copy(deg_v, deg_sh.at[s, pl.ds(0, _N)])
    plsc.subcore_barrier()

    pltpu.sync_copy(deg_sh.at[:, pl.ds(s * _SLICE, _SLICE)], red_v)

    def _dred(j, carry):
        acc = red_v[0, pl.ds(j * 16, 16)]
        for k in range(1, _NT):
            acc = acc + red_v[k, pl.ds(j * 16, 16)]
        slc_v[pl.ds(j * 16, 16)] = acc
        return carry
    lax.fori_loop(0, _SLICE // 16, _dred, None)
    pltpu.sync_copy(slc_v, degp_hbm.at[c, pl.ds(s * _SLICE, _SLICE)])


_deg_sc = functools.partial(
    pl.kernel,
    out_type=jax.ShapeDtypeStruct((_NC, _NPAD), jnp.float32),
    mesh=_SC_MESH,
    compiler_params=_SC_PARAMS,
    scratch_types=[
        pltpu.VMEM((_PIECE,), jnp.int32),
        pltpu.VMEM((_PIECE,), jnp.int32),
        pltpu.VMEM((_PIECE,), jnp.float32),
        pltpu.VMEM((_PIECE,), jnp.int32),
        pltpu.VMEM((_PIECE,), jnp.int32),
        pltpu.VMEM((_PIECE,), jnp.float32),
        pltpu.VMEM((_N,), jnp.float32),
        pltpu.VMEM((_NT, _SLICE), jnp.float32),
        pltpu.VMEM((_SLICE,), jnp.float32),
        pltpu.SemaphoreType.DMA,
        pltpu.SemaphoreType.DMA,
        pltpu.VMEM_SHARED((_NT, _NPAD), jnp.float32),
    ],
)(_deg_body)


def _msg_body(ei_hbm, w_hbm, z1_hbm, degp_hbm, t_hbm,
              sA, dA, wA, sB, dB, wB, z1_v, dinv_v, t_v, red_v, slc_v, dp_v,
              z1_sem, semA, semB, dinv_sh, t_sh):
    c = lax.axis_index("c")
    s = lax.axis_index("s")
    bufs = ((sA, dA, wA, semA), (sB, dB, wB, semB))
    start_piece = _piece_starter(ei_hbm, w_hbm, bufs)
    base = s * _CHUNK + c * _HALF

    pend = start_piece(0, base)
    z1_cp = pltpu.make_async_copy(z1_hbm, z1_v, z1_sem)
    z1_cp.start()

    pltpu.sync_copy(degp_hbm.at[:, pl.ds(s * _SLICE, _SLICE)], dp_v)

    def _dinv(j, carry):
        acc = dp_v[0, pl.ds(j * 16, 16)] + dp_v[1, pl.ds(j * 16, 16)]
        slc_v[pl.ds(j * 16, 16)] = _rsqrt16(acc)
        return carry
    lax.fori_loop(0, _SLICE // 16, _dinv, None)
    pltpu.sync_copy(slc_v, dinv_sh.at[pl.ds(s * _SLICE, _SLICE)])

    def _zero(i, carry):
        for u in range(5):
            t_v[pl.ds(i * 80 + u * 16, 16)] = jnp.zeros((16,), jnp.float32)
        return carry
    lax.fori_loop(0, (_B * _N) // 80, _zero, None)

    plsc.subcore_barrier()
    pltpu.sync_copy(dinv_sh.at[pl.ds(0, _N)], dinv_v)
    z1_cp.wait()

    def _make_msg(q):
        sv, dv, wv, _ = bufs[q]

        def _msg(i, carry):
            for u in range(5):
                off = i * 80 + u * 16
                a = sv[pl.ds(off, 16)]
                b = dv[pl.ds(off, 16)]
                w16 = wv[pl.ds(off, 16)]
                dsrc = plsc.load_gather(dinv_v, [a])
                ddst = plsc.load_gather(dinv_v, [b])
                ce = jnp.where(a != b, -(dsrc * w16 * ddst), 0.0)
                g0 = plsc.load_gather(z1_v, [a])
                g1 = plsc.load_gather(z1_v, [a + _N])
                plsc.addupdate_scatter(t_v, [b], ce * g0)
                plsc.addupdate_scatter(t_v, [b + _N], ce * g1)
            return carry
        return _msg

    msgs = (_make_msg(0), _make_msg(1))
    for p in range(_NPP):
        q = p & 1
        cur = pend
        if p + 1 < _NPP:
            pend = start_piece(q ^ 1, base + (p + 1) * _PIECE)
        for cp in cur:
            cp.wait()
        lax.fori_loop(0, _PIECE // 80, msgs[q], None)

    pltpu.sync_copy(t_v.at[pl.ds(0, _N)], t_sh.at[s, 0, pl.ds(0, _N)])
    pltpu.sync_copy(t_v.at[pl.ds(_N, _N)], t_sh.at[s, 1, pl.ds(0, _N)])
    plsc.subcore_barrier()

    for b in range(_B):
        pltpu.sync_copy(t_sh.at[:, b, pl.ds(s * _SLICE, _SLICE)], red_v)

        def _tred(j, carry):
            acc = red_v[0, pl.ds(j * 16, 16)]
            for k in range(1, _NT):
                acc = acc + red_v[k, pl.ds(j * 16, 16)]
            slc_v[pl.ds(j * 16, 16)] = acc
            return carry
        lax.fori_loop(0, _SLICE // 16, _tred, None)
        pltpu.sync_copy(slc_v, t_hbm.at[b, c, pl.ds(s * _SLICE, _SLICE)])


_msg_sc = functools.partial(
    pl.kernel,
    out_type=jax.ShapeDtypeStruct((_B, _NC, _NPAD), jnp.float32),
    mesh=_SC_MESH,
    compiler_params=_SC_PARAMS,
    scratch_types=[
        pltpu.VMEM((_PIECE,), jnp.int32),
        pltpu.VMEM((_PIECE,), jnp.int32),
        pltpu.VMEM((_PIECE,), jnp.float32),
        pltpu.VMEM((_PIECE,), jnp.int32),
        pltpu.VMEM((_PIECE,), jnp.int32),
        pltpu.VMEM((_PIECE,), jnp.float32),
        pltpu.VMEM((_B * _N,), jnp.float32),
        pltpu.VMEM((_N,), jnp.float32),
        pltpu.VMEM((_B * _N,), jnp.float32),
        pltpu.VMEM((_NT, _SLICE), jnp.float32),
        pltpu.VMEM((_SLICE,), jnp.float32),
        pltpu.VMEM((_NC, _SLICE), jnp.float32),
        pltpu.SemaphoreType.DMA,
        pltpu.SemaphoreType.DMA,
        pltpu.SemaphoreType.DMA,
        pltpu.VMEM_SHARED((_NPAD,), jnp.float32),
        pltpu.VMEM_SHARED((_NT, _B, _NPAD), jnp.float32),
    ],
)(_msg_body)


def _mv_body(x_ref, w0_ref, w1_ref, z0_ref, z1_ref):
    xb = x_ref[...]
    z0 = jnp.dot(xb, w0_ref[...], preferred_element_type=jnp.float32)
    z1 = jnp.dot(xb, w1_ref[...], preferred_element_type=jnp.float32)
    z0_ref[0, 0] = z0[:, 0]
    z1_ref[0, 0] = z1[:, 0]


def _matvec(x2, w0, w1):
    return pl.pallas_call(
        _mv_body,
        grid=(10,),
        in_specs=[
            pl.BlockSpec((2000, _F), lambda i: (i, 0)),
            pl.BlockSpec((_F, 1), lambda i: (0, 0)),
            pl.BlockSpec((_F, 1), lambda i: (0, 0)),
        ],
        out_specs=[
            pl.BlockSpec((1, 1, 2000), lambda i: (i, 0, 0)),
            pl.BlockSpec((1, 1, 2000), lambda i: (i, 0, 0)),
        ],
        out_shape=[
            jax.ShapeDtypeStruct((10, 1, 2000), jnp.float32),
            jax.ShapeDtypeStruct((10, 1, 2000), jnp.float32),
        ],
    )(x2, w0, w1)


def _y_body(z0_ref, t_ref, b_ref, y_ref):
    sv = z0_ref[0, 0] + t_ref[0, 0, :_N] + t_ref[0, 1, :_N] + b_ref[0]
    y_ref[0, 0] = jax.nn.sigmoid(sv)


def _sigmoid_y(z0, t, bias):
    return pl.pallas_call(
        _y_body,
        grid=(_B,),
        in_specs=[
            pl.BlockSpec((1, 1, _N), lambda b: (b, 0, 0)),
            pl.BlockSpec((1, _NC, _NPAD), lambda b: (b, 0, 0)),
            pl.BlockSpec((1,), lambda b: (0,)),
        ],
        out_specs=pl.BlockSpec((1, 1, _N), lambda b: (b, 0, 0)),
        out_shape=jax.ShapeDtypeStruct((_B, 1, _N), jnp.float32),
    )(z0, t, bias)


def kernel(x, edge_index, edge_w, W0, W1, bias):
    ei = edge_index.astype(jnp.int32)
    x2 = x.reshape(_B * _N, _F)
    degp = _deg_sc(ei, edge_w)
    z0, z1 = _matvec(x2, W0, W1)
    t = _msg_sc(ei, edge_w, z1.reshape(_B * _N), degp)
    y = _sigmoid_y(z0.reshape(_B, 1, _N), t, bias)
    return jnp.concatenate([x, y.reshape(_B, _N, 1)], axis=-1)


# speedup vs baseline: 2.4040x; 1.1340x over previous
import functools

import jax
import jax.numpy as jnp
from jax import lax
from jax.experimental import pallas as pl
from jax.experimental.pallas import tpu as pltpu
from jax.experimental.pallas import tpu_sc as plsc

_N = 10000
_E = 320000
_B = 2
_F = 128
_NT = 16
_NC = 2
_CHUNK = _E // _NT
_HALF = _CHUNK // _NC
_PIECE = 2000
_NPP = _HALF // _PIECE
_NPAD = 10240
_SLICE = _NPAD // _NT

_SC_PARAMS = pltpu.CompilerParams(use_tc_tiling_on_sc=False,
                                  needs_layout_passes=False)
_SC_MESH = plsc.VectorSubcoreMesh(core_axis_name="c", subcore_axis_name="s")


def _rsqrt16(d):
    i = plsc.bitcast(d, jnp.int32)
    i = jnp.int32(0x5F3759DF) - (i >> 1)
    y = plsc.bitcast(i, jnp.float32)
    for _ in range(3):
        y = y * (1.5 - 0.5 * d * y * y)
    return jnp.where(d > 0.0, y, 0.0)


def _piece_starter(ei_hbm, w_hbm, bufs):
    def start_piece(q, off):
        sv, dv, wv, sem = bufs[q]
        cps = (pltpu.make_async_copy(ei_hbm.at[0, pl.ds(off, _PIECE)], sv, sem),
               pltpu.make_async_copy(ei_hbm.at[1, pl.ds(off, _PIECE)], dv, sem),
               pltpu.make_async_copy(w_hbm.at[pl.ds(off, _PIECE)], wv, sem))
        for cp in cps:
            cp.start()
        return cps
    return start_piece


def _deg_body(ei_hbm, w_hbm, degp_hbm,
              sA, dA, wA, sB, dB, wB, deg_v, red_v, slc_v, semA, semB,
              deg_sh):
    c = lax.axis_index("c")
    s = lax.axis_index("s")
    bufs = ((sA, dA, wA, semA), (sB, dB, wB, semB))
    start_piece = _piece_starter(ei_hbm, w_hbm, bufs)
    base = s * _CHUNK + c * _HALF

    pend = start_piece(0, base)

    def _zero(i, carry):
        for u in range(5):
            deg_v[pl.ds(i * 80 + u * 16, 16)] = jnp.zeros((16,), jnp.float32)
        return carry
    lax.fori_loop(0, _N // 80, _zero, None)

    def _make_deg(q):
        sv, dv, wv, _ = bufs[q]

        def _deg(i, carry):
            for u in range(5):
                off = i * 80 + u * 16
                a = sv[pl.ds(off, 16)]
                b = dv[pl.ds(off, 16)]
                w16 = wv[pl.ds(off, 16)]
                plsc.addupdate_scatter(deg_v, [a], jnp.where(a != b, w16, 0.0))
            return carry
        return _deg

    degs = (_make_deg(0), _make_deg(1))
    for p in range(_NPP):
        q = p & 1
        cur = pend
        if p + 1 < _NPP:
            pend = start_piece(q ^ 1, base + (p + 1) * _PIECE)
        for cp in cur:
            cp.wait()
        lax.fori_loop(0, _PIECE // 80, degs[q], None)

    pltpu.sync_copy(deg_v, deg_sh.at[s, pl.ds(0, _N)])
    plsc.subcore_barrier()

    pltpu.sync_copy(deg_sh.at[:, pl.ds(s * _SLICE, _SLICE)], red_v)

    def _dred(j, carry):
        acc = red_v[0, pl.ds(j * 16, 16)]
        for k in range(1, _NT):
            acc = acc + red_v[k, pl.ds(j * 16, 16)]
        slc_v[pl.ds(j * 16, 16)] = acc
        return carry
    lax.fori_loop(0, _SLICE // 16, _dred, None)
    pltpu.sync_copy(slc_v, degp_hbm.at[c, pl.ds(s * _SLICE, _SLICE)])


_deg_sc = functools.partial(
    pl.kernel,
    out_type=jax.ShapeDtypeStruct((_NC, _NPAD), jnp.float32),
    mesh=_SC_MESH,
    compiler_params=_SC_PARAMS,
    scratch_types=[
        pltpu.VMEM((_PIECE,), jnp.int32),
        pltpu.VMEM((_PIECE,), jnp.int32),
        pltpu.VMEM((_PIECE,), jnp.float32),
        pltpu.VMEM((_PIECE,), jnp.int32),
        pltpu.VMEM((_PIECE,), jnp.int32),
        pltpu.VMEM((_PIECE,), jnp.float32),
        pltpu.VMEM((_N,), jnp.float32),
        pltpu.VMEM((_NT, _SLICE), jnp.float32),
        pltpu.VMEM((_SLICE,), jnp.float32),
        pltpu.SemaphoreType.DMA,
        pltpu.SemaphoreType.DMA,
        pltpu.VMEM_SHARED((_NT, _NPAD), jnp.float32),
    ],
)(_deg_body)


def _msg_body(ei_hbm, w_hbm, z1_hbm, degp_hbm, t_hbm,
              sA, dA, wA, sB, dB, wB, z1_v, dinv_v, t_v, red_v, slc_v, dp_v,
              z1_sem, semA, semB, dinv_sh, t_sh):
    c = lax.axis_index("c")
    s = lax.axis_index("s")
    bufs = ((sA, dA, wA, semA), (sB, dB, wB, semB))
    start_piece = _piece_starter(ei_hbm, w_hbm, bufs)
    base = s * _CHUNK + c * _HALF

    pend = start_piece(0, base)
    z1_cp = pltpu.make_async_copy(z1_hbm, z1_v, z1_sem)
    z1_cp.start()

    pltpu.sync_copy(degp_hbm.at[:, pl.ds(s * _SLICE, _SLICE)], dp_v)

    def _dinv(j, carry):
        acc = dp_v[0, pl.ds(j * 16, 16)] + dp_v[1, pl.ds(j * 16, 16)]
        slc_v[pl.ds(j * 16, 16)] = _rsqrt16(acc)
        return carry
    lax.fori_loop(0, _SLICE // 16, _dinv, None)
    pltpu.sync_copy(slc_v, dinv_sh.at[pl.ds(s * _SLICE, _SLICE)])

    def _zero(i, carry):
        for u in range(5):
            t_v[pl.ds(i * 80 + u * 16, 16)] = jnp.zeros((16,), jnp.float32)
        return carry
    lax.fori_loop(0, (_B * _N) // 80, _zero, None)

    plsc.subcore_barrier()
    pltpu.sync_copy(dinv_sh.at[pl.ds(0, _N)], dinv_v)
    z1_cp.wait()

    def _make_msg(q):
        sv, dv, wv, _ = bufs[q]

        def _msg(i, carry):
            for u in range(5):
                off = i * 80 + u * 16
                a = sv[pl.ds(off, 16)]
                b = dv[pl.ds(off, 16)]
                w16 = wv[pl.ds(off, 16)]
                dsrc = plsc.load_gather(dinv_v, [a])
                ddst = plsc.load_gather(dinv_v, [b])
                ce = jnp.where(a != b, -(dsrc * w16 * ddst), 0.0)
                g0 = plsc.load_gather(z1_v, [a])
                g1 = plsc.load_gather(z1_v, [a + _N])
                plsc.addupdate_scatter(t_v, [b], ce * g0)
                plsc.addupdate_scatter(t_v, [b + _N], ce * g1)
            return carry
        return _msg

    msgs = (_make_msg(0), _make_msg(1))
    for p in range(_NPP):
        q = p & 1
        cur = pend
        if p + 1 < _NPP:
            pend = start_piece(q ^ 1, base + (p + 1) * _PIECE)
        for cp in cur:
            cp.wait()
        lax.fori_loop(0, _PIECE // 80, msgs[q], None)

    pltpu.sync_copy(t_v.at[pl.ds(0, _N)], t_sh.at[s, 0, pl.ds(0, _N)])
    pltpu.sync_copy(t_v.at[pl.ds(_N, _N)], t_sh.at[s, 1, pl.ds(0, _N)])
    plsc.subcore_barrier()

    for b in range(_B):
        pltpu.sync_copy(t_sh.at[:, b, pl.ds(s * _SLICE, _SLICE)], red_v)

        def _tred(j, carry):
            acc = red_v[0, pl.ds(j * 16, 16)]
            for k in range(1, _NT):
                acc = acc + red_v[k, pl.ds(j * 16, 16)]
            slc_v[pl.ds(j * 16, 16)] = acc
            return carry
        lax.fori_loop(0, _SLICE // 16, _tred, None)
        pltpu.sync_copy(slc_v, t_hbm.at[b, c, pl.ds(s * _SLICE, _SLICE)])


_msg_sc = functools.partial(
    pl.kernel,
    out_type=jax.ShapeDtypeStruct((_B, _NC, _NPAD), jnp.float32),
    mesh=_SC_MESH,
    compiler_params=_SC_PARAMS,
    scratch_types=[
        pltpu.VMEM((_PIECE,), jnp.int32),
        pltpu.VMEM((_PIECE,), jnp.int32),
        pltpu.VMEM((_PIECE,), jnp.float32),
        pltpu.VMEM((_PIECE,), jnp.int32),
        pltpu.VMEM((_PIECE,), jnp.int32),
        pltpu.VMEM((_PIECE,), jnp.float32),
        pltpu.VMEM((_B * _N,), jnp.float32),
        pltpu.VMEM((_N,), jnp.float32),
        pltpu.VMEM((_B * _N,), jnp.float32),
        pltpu.VMEM((_NT, _SLICE), jnp.float32),
        pltpu.VMEM((_SLICE,), jnp.float32),
        pltpu.VMEM((_NC, _SLICE), jnp.float32),
        pltpu.SemaphoreType.DMA,
        pltpu.SemaphoreType.DMA,
        pltpu.SemaphoreType.DMA,
        pltpu.VMEM_SHARED((_NPAD,), jnp.float32),
        pltpu.VMEM_SHARED((_NT, _B, _NPAD), jnp.float32),
    ],
)(_msg_body)


def _mv_body(x_ref, w0_ref, w1_ref, z0_ref, z1_ref):
    xb = x_ref[...]
    wcat = jnp.concatenate([w0_ref[...], w1_ref[...]], axis=1)
    zt = lax.dot_general(wcat, xb, dimension_numbers=(((0,), (1,)), ((), ())),
                         preferred_element_type=jnp.float32)
    z0_ref[0, 0] = zt[0]
    z1_ref[0, 0] = zt[1]


def _matvec(x2, w0, w1):
    return pl.pallas_call(
        _mv_body,
        grid=(10,),
        in_specs=[
            pl.BlockSpec((2000, _F), lambda i: (i, 0)),
            pl.BlockSpec((_F, 1), lambda i: (0, 0)),
            pl.BlockSpec((_F, 1), lambda i: (0, 0)),
        ],
        out_specs=[
            pl.BlockSpec((1, 1, 2000), lambda i: (i, 0, 0)),
            pl.BlockSpec((1, 1, 2000), lambda i: (i, 0, 0)),
        ],
        out_shape=[
            jax.ShapeDtypeStruct((10, 1, 2000), jnp.float32),
            jax.ShapeDtypeStruct((10, 1, 2000), jnp.float32),
        ],
    )(x2, w0, w1)


def _y_body(z0_ref, t_ref, b_ref, y_ref):
    sv = z0_ref[0, 0] + t_ref[0, 0, :_N] + t_ref[0, 1, :_N] + b_ref[0]
    y_ref[0, 0] = jax.nn.sigmoid(sv)


def _sigmoid_y(z0, t, bias):
    return pl.pallas_call(
        _y_body,
        grid=(_B,),
        in_specs=[
            pl.BlockSpec((1, 1, _N), lambda b: (b, 0, 0)),
            pl.BlockSpec((1, _NC, _NPAD), lambda b: (b, 0, 0)),
            pl.BlockSpec((1,), lambda b: (0,)),
        ],
        out_specs=pl.BlockSpec((1, 1, _N), lambda b: (b, 0, 0)),
        out_shape=jax.ShapeDtypeStruct((_B, 1, _N), jnp.float32),
    )(z0, t, bias)


def kernel(x, edge_index, edge_w, W0, W1, bias):
    ei = edge_index.astype(jnp.int32)
    x2 = x.reshape(_B * _N, _F)
    degp = _deg_sc(ei, edge_w)
    z0, z1 = _matvec(x2, W0, W1)
    t = _msg_sc(ei, edge_w, z1.reshape(_B * _N), degp)
    y = _sigmoid_y(z0.reshape(_B, 1, _N), t, bias)
    return jnp.concatenate([x, y.reshape(_B, _N, 1)], axis=-1)
